# baseline (speedup 1.0000x reference)
import jax
import jax.numpy as jnp
from jax import lax
from jax.experimental import pallas as pl
from jax.experimental.pallas import tpu as pltpu
from jax.experimental.pallas import tpu_sc as plsc

NODE_DIM = 128
EDGE_DIM = 16
N_NODES = 10000
N_EDGES = 320000

NC = 2
NS = 16
NW = NC * NS
NSPLIT = 2
E_CHU = N_EDGES // NSPLIT
PER_W = E_CHU // NW
CHB = 64
NFULL = PER_W // CHB
TAIL_E = PER_W - NFULL * CHB
TAIL_E_OFF = NFULL * CHB
G_SLOTS = 6
G_AHEAD = 3

ROWS_PER_SUB = 624
TAIL_ROWS = N_NODES - NS * ROWS_PER_SUB
TAIL_OFF = NS * ROWS_PER_SUB

G_DEPTH = 3

S_CH = 40
S_NCHUNK = PER_W // S_CH
S_DEPTH = 4

BE = 4000
BN = 1000


def _mesh():
    return plsc.VectorSubcoreMesh(
        core_axis_name="c", subcore_axis_name="s", num_cores=NC, num_subcores=NS)


_NEG_LOG2E = -1.4426950408889634
_LN2 = 0.6931471805599453


def _softplus(v):
    e = jnp.exp2(jnp.abs(v) * _NEG_LOG2E)
    return jnp.maximum(v, 0.0) + jnp.log2(1.0 + e) * _LN2


def _gather_body(x_hbm, row_m, row_t, col_m, col_t, xr_hbm, xc_hbm,
                 idx_rm, idx_rt, idx_cm, idx_ct,
                 bufs_r, bufs_c, brt, bct,
                 gsems_r, gsems_c, wsems_r, wsems_c, srt, sct):
    c = lax.axis_index("c")
    s = lax.axis_index("s")
    wid = s * NC + c
    base = wid * PER_W
    pltpu.sync_copy(row_m.at[wid], idx_rm)
    pltpu.sync_copy(row_t.at[wid], idx_rt)
    pltpu.sync_copy(col_m.at[wid], idx_cm)
    pltpu.sync_copy(col_t.at[wid], idx_ct)

    def fire_gather(j, u):
        pltpu.async_copy(x_hbm.at[idx_rm.at[j]], bufs_r[u], gsems_r[u])
        pltpu.async_copy(x_hbm.at[idx_cm.at[j]], bufs_c[u], gsems_c[u])

    def wait_gather(j, u):
        pltpu.make_async_copy(x_hbm.at[idx_rm.at[j]], bufs_r[u],
                              gsems_r[u]).wait()
        pltpu.make_async_copy(x_hbm.at[idx_cm.at[j]], bufs_c[u],
                              gsems_c[u]).wait()

    def fire_write(j, u):
        off = base + j * CHB
        pltpu.async_copy(bufs_r[u], xr_hbm.at[pl.ds(off, CHB)], wsems_r[u])
        pltpu.async_copy(bufs_c[u], xc_hbm.at[pl.ds(off, CHB)], wsems_c[u])

    def wait_write(j, u):
        off = base + j * CHB
        pltpu.make_async_copy(bufs_r[u], xr_hbm.at[pl.ds(off, CHB)],
                              wsems_r[u]).wait()
        pltpu.make_async_copy(bufs_c[u], xc_hbm.at[pl.ds(off, CHB)],
                              wsems_c[u]).wait()

    for k in range(G_AHEAD):
        fire_gather(k, k)

    ngrp = NFULL // G_SLOTS

    def body(t, carry):
        for u in range(G_SLOTS):
            j = t * G_SLOTS + u
            ua = (u + G_AHEAD) % G_SLOTS

            @pl.when(j + G_AHEAD < NFULL)
            def _():
                @pl.when(j >= G_AHEAD)
                def _():
                    wait_write(j - G_AHEAD, ua)

                fire_gather(j + G_AHEAD, ua)

            wait_gather(j, u)
            fire_write(j, u)
        return carry

    lax.fori_loop(0, ngrp, body, 0)
    for j in range(NFULL - G_SLOTS, NFULL):
        wait_write(j, j % G_SLOTS)

    cp_r = pltpu.async_copy(x_hbm.at[idx_rt.at[0]], brt, srt)
    cp_c = pltpu.async_copy(x_hbm.at[idx_ct.at[0]], bct, sct)
    cp_r.wait()
    cp_c.wait()
    off = base + TAIL_E_OFF
    pltpu.sync_copy(brt, xr_hbm.at[pl.ds(off, TAIL_E)])
    pltpu.sync_copy(bct, xc_hbm.at[pl.ds(off, TAIL_E)])


@jax.jit
def _gather(x, row_m, row_t, col_m, col_t):
    def wrapped(x_hbm, row_m_, row_t_, col_m_, col_t_, xr_hbm, xc_hbm,
                idx_rm, idx_rt, idx_cm, idx_ct, *rest):
        bufs_r = rest[0:G_SLOTS]
        bufs_c = rest[G_SLOTS:2 * G_SLOTS]
        brt, bct = rest[2 * G_SLOTS:2 * G_SLOTS + 2]
        sbase = 2 * G_SLOTS + 2
        gsems_r = rest[sbase:sbase + G_SLOTS]
        gsems_c = rest[sbase + G_SLOTS:sbase + 2 * G_SLOTS]
        wsems_r = rest[sbase + 2 * G_SLOTS:sbase + 3 * G_SLOTS]
        wsems_c = rest[sbase + 3 * G_SLOTS:sbase + 4 * G_SLOTS]
        srt, sct = rest[sbase + 4 * G_SLOTS:sbase + 4 * G_SLOTS + 2]
        _gather_body(x_hbm, row_m_, row_t_, col_m_, col_t_, xr_hbm, xc_hbm,
                     idx_rm, idx_rt, idx_cm, idx_ct,
                     bufs_r, bufs_c, brt, bct,
                     gsems_r, gsems_c, wsems_r, wsems_c, srt, sct)

    f = pl.kernel(
        wrapped,
        out_type=(
            jax.ShapeDtypeStruct((E_CHU, NODE_DIM), jnp.float32),
            jax.ShapeDtypeStruct((E_CHU, NODE_DIM), jnp.float32),
        ),
        mesh=_mesh(),
        scratch_types=(
            [pltpu.VMEM((NFULL, CHB), jnp.int32),
             pltpu.VMEM((1, TAIL_E), jnp.int32)] * 2
            + [pltpu.VMEM((CHB, NODE_DIM), jnp.float32)] * (2 * G_SLOTS)
            + [pltpu.VMEM((TAIL_E, NODE_DIM), jnp.float32)] * 2
            + [pltpu.SemaphoreType.DMA] * (4 * G_SLOTS + 2)
        ),
    )
    return f(x, row_m, row_t, col_m, col_t)


def _scatter_body(emb_hbm, col_s, zeros_hbm, out_hbm,
                  idx_m, b0, b1, b2, b3, shared,
                  s0, s1, s2, s3):
    c = lax.axis_index("c")
    s = lax.axis_index("s")
    wid = s * NC + c
    base = wid * PER_W
    r0 = s * ROWS_PER_SUB
    pltpu.sync_copy(zeros_hbm, shared.at[pl.ds(r0, ROWS_PER_SUB)])

    @pl.when(s == NS - 1)
    def _():
        pltpu.sync_copy(zeros_hbm.at[pl.ds(0, TAIL_ROWS)],
                        shared.at[pl.ds(TAIL_OFF, TAIL_ROWS)])

    pltpu.sync_copy(col_s.at[wid], idx_m)
    plsc.subcore_barrier()

    bufs = ((b0, s0), (b1, s1), (b2, s2), (b3, s3))

    def fire(j, k):
        b, sm = bufs[k]
        pltpu.async_copy(emb_hbm.at[pl.ds(base + j * S_CH, S_CH)], b, sm)

    def drain_add(j, k):
        b, sm = bufs[k]
        pltpu.make_async_copy(emb_hbm.at[pl.ds(base + j * S_CH, S_CH)], b,
                              sm).wait()
        pltpu.sync_copy(b, shared.at[idx_m.at[j]], add=True)

    for k in range(S_DEPTH):
        fire(k, k)

    ngrp = S_NCHUNK // S_DEPTH

    def body(t, carry):
        for u in range(S_DEPTH):
            j = t * S_DEPTH + u
            drain_add(j, u)

            @pl.when(j + S_DEPTH < S_NCHUNK)
            def _():
                fire(j + S_DEPTH, u)
        return carry

    lax.fori_loop(0, ngrp, body, 0)
    for u in range(S_NCHUNK - ngrp * S_DEPTH):
        drain_add(ngrp * S_DEPTH + u, u)

    plsc.subcore_barrier()
    pltpu.sync_copy(shared.at[pl.ds(r0, ROWS_PER_SUB)],
                    out_hbm.at[c, pl.ds(r0, ROWS_PER_SUB)])

    @pl.when(s == NS - 1)
    def _():
        pltpu.sync_copy(shared.at[pl.ds(TAIL_OFF, TAIL_ROWS)],
                        out_hbm.at[c, pl.ds(TAIL_OFF, TAIL_ROWS)])


@jax.jit
def _scatter(emb, col_s, zeros):
    f = pl.kernel(
        _scatter_body,
        out_type=jax.ShapeDtypeStruct((NC, N_NODES, NODE_DIM), jnp.float32),
        mesh=_mesh(),
        scratch_types=(
            [pltpu.VMEM((S_NCHUNK, S_CH), jnp.int32)]
            + [pltpu.VMEM((S_CH, NODE_DIM), jnp.float32)] * S_DEPTH
            + [pltpu.VMEM_SHARED((N_NODES, NODE_DIM), jnp.float32)]
            + [pltpu.SemaphoreType.DMA] * S_DEPTH
        ),
    )
    return f(emb, col_s, zeros)


def _edge_mlp_body(xr, xc, ea, w1a, w1b, w1c, b1, w2, b2, out):
    bf = jnp.bfloat16
    acc = jnp.dot(xr[...].astype(bf), w1a[...],
                  preferred_element_type=jnp.float32)
    acc += jnp.dot(xc[...].astype(bf), w1b[...],
                   preferred_element_type=jnp.float32)
    acc += jnp.dot(ea[...].astype(bf), w1c[...],
                   preferred_element_type=jnp.float32)
    acc += b1[...]
    h = _softplus(acc)
    o = jnp.dot(h.astype(bf), w2[...],
                preferred_element_type=jnp.float32) + b2[...]
    out[...] = _softplus(o)


@jax.jit
def _edge_mlp(xr, xc, ea, w1a, w1b, w1c, b1, w2, b2):
    nblk = E_CHU // BE
    full = lambda shape: pl.BlockSpec(shape, lambda i: (0, 0))
    return pl.pallas_call(
        _edge_mlp_body,
        grid=(nblk,),
        in_specs=[
            pl.BlockSpec((BE, NODE_DIM), lambda i: (i, 0)),
            pl.BlockSpec((BE, NODE_DIM), lambda i: (i, 0)),
            pl.BlockSpec((BE, EDGE_DIM), lambda i: (i, 0)),
            full((NODE_DIM, 2 * NODE_DIM)),
            full((NODE_DIM, 2 * NODE_DIM)),
            full((EDGE_DIM, 2 * NODE_DIM)),
            full((1, 2 * NODE_DIM)),
            full((2 * NODE_DIM, NODE_DIM)),
            full((1, NODE_DIM)),
        ],
        out_specs=pl.BlockSpec((BE, NODE_DIM), lambda i: (i, 0)),
        out_shape=jax.ShapeDtypeStruct((E_CHU, NODE_DIM), jnp.float32),
        compiler_params=pltpu.CompilerParams(
            dimension_semantics=("parallel",)),
    )(xr, xc, ea, w1a, w1b, w1c, b1, w2, b2)


def _node_body(x, a0, a1, a2, a3, w3a, w3b, b3, out):
    ag = (a0[...] + a1[...]) + (a2[...] + a3[...])
    o = jnp.dot(x[...], w3a[...], preferred_element_type=jnp.float32)
    o += jnp.dot(ag, w3b[...], preferred_element_type=jnp.float32)
    o += b3[...]
    out[...] = _softplus(o) + x[...]


@jax.jit
def _node(x, a0, a1, a2, a3, w3a, w3b, b3):
    nblk = N_NODES // BN
    full = lambda shape: pl.BlockSpec(shape, lambda i: (0, 0))
    blk = pl.BlockSpec((BN, NODE_DIM), lambda i: (i, 0))
    return pl.pallas_call(
        _node_body,
        grid=(nblk,),
        in_specs=[
            blk, blk, blk, blk, blk,
            full((NODE_DIM, NODE_DIM)),
            full((NODE_DIM, NODE_DIM)),
            full((1, NODE_DIM)),
        ],
        out_specs=pl.BlockSpec((BN, NODE_DIM), lambda i: (i, 0)),
        out_shape=jax.ShapeDtypeStruct((N_NODES, NODE_DIM), jnp.float32),
        compiler_params=pltpu.CompilerParams(
            dimension_semantics=("parallel",)),
    )(x, a0, a1, a2, a3, w3a, w3b, b3)


def _split_idx(v):
    flat = v.astype(jnp.int32).reshape(NSPLIT, NW, PER_W)
    main = flat[:, :, :TAIL_E_OFF].reshape(NSPLIT, NW, NFULL, CHB)
    tail = flat[:, :, TAIL_E_OFF:].reshape(NSPLIT, NW, 1, TAIL_E)
    return main, tail


def kernel(x, edge_index, edge_attr, W1, b1, W2, b2, W3, b3):
    row_m, row_t = _split_idx(edge_index[0])
    col_m, col_t = _split_idx(edge_index[1])
    col_s = edge_index[1].astype(jnp.int32).reshape(NSPLIT, NW, S_NCHUNK, S_CH)
    bf = jnp.bfloat16
    w1a = W1[:NODE_DIM].astype(bf)
    w1b = W1[NODE_DIM:2 * NODE_DIM].astype(bf)
    w1c = W1[2 * NODE_DIM:].astype(bf)
    b1r = b1.reshape(1, -1)
    w2 = W2.astype(bf)
    b2r = b2.reshape(1, -1)
    zeros = jnp.zeros((ROWS_PER_SUB, NODE_DIM), jnp.float32)

    parts = []
    for t in range(NSPLIT):
        xr, xc = _gather(x, row_m[t], row_t[t], col_m[t], col_t[t])
        emb = _edge_mlp(xr, xc, edge_attr[t * E_CHU:(t + 1) * E_CHU],
                        w1a, w1b, w1c, b1r, w2, b2r)
        parts.append(_scatter(emb, col_s[t], zeros))
    return _node(x, parts[0][0], parts[0][1], parts[1][0], parts[1][1],
                 W3[:NODE_DIM], W3[NODE_DIM:], b3.reshape(1, -1))


# speedup vs baseline: 1.0328x; 1.0328x over previous
import jax
import jax.numpy as jnp
from jax import lax
from jax.experimental import pallas as pl
from jax.experimental.pallas import tpu as pltpu
from jax.experimental.pallas import tpu_sc as plsc

NODE_DIM = 128
EDGE_DIM = 16
N_NODES = 10000
N_EDGES = 320000

NC = 2
NS = 16
NW = NC * NS
NSPLIT = 2
E_CHU = N_EDGES // NSPLIT
PER_W = E_CHU // NW
CHB = 64
NFULL = PER_W // CHB
TAIL_E = PER_W - NFULL * CHB
TAIL_E_OFF = NFULL * CHB
G_SLOTS = 6
G_AHEAD = 3

ROWS_PER_SUB = 624
TAIL_ROWS = N_NODES - NS * ROWS_PER_SUB
TAIL_OFF = NS * ROWS_PER_SUB

G_DEPTH = 3

S_CH = 40
S_NCHUNK = PER_W // S_CH
S_DEPTH = 4

BE = 4000
BN = 1000


def _mesh():
    return plsc.VectorSubcoreMesh(
        core_axis_name="c", subcore_axis_name="s", num_cores=NC, num_subcores=NS)


_NEG_LOG2E = -1.4426950408889634
_LN2 = 0.6931471805599453


def _softplus(v):
    e = jnp.exp2(jnp.abs(v) * _NEG_LOG2E)
    return jnp.maximum(v, 0.0) + jnp.log2(1.0 + e) * _LN2


def _gather_body(x_hbm, row_m, row_t, col_m, col_t, xr_hbm, xc_hbm,
                 idx_rm, idx_rt, idx_cm, idx_ct,
                 bufs_r, bufs_c, brt, bct,
                 gsems_r, gsems_c, wsems_r, wsems_c, srt, sct):
    c = lax.axis_index("c")
    s = lax.axis_index("s")
    wid = s * NC + c
    base = wid * PER_W
    pltpu.sync_copy(row_m.at[wid], idx_rm)
    pltpu.sync_copy(row_t.at[wid], idx_rt)
    pltpu.sync_copy(col_m.at[wid], idx_cm)
    pltpu.sync_copy(col_t.at[wid], idx_ct)

    def fire_gather(j, u):
        pltpu.async_copy(x_hbm.at[idx_rm.at[j]], bufs_r[u], gsems_r[u])
        pltpu.async_copy(x_hbm.at[idx_cm.at[j]], bufs_c[u], gsems_c[u])

    def wait_gather(j, u):
        pltpu.make_async_copy(x_hbm.at[idx_rm.at[j]], bufs_r[u],
                              gsems_r[u]).wait()
        pltpu.make_async_copy(x_hbm.at[idx_cm.at[j]], bufs_c[u],
                              gsems_c[u]).wait()

    def fire_write(j, u):
        off = base + j * CHB
        pltpu.async_copy(bufs_r[u], xr_hbm.at[pl.ds(off, CHB)], wsems_r[u])
        pltpu.async_copy(bufs_c[u], xc_hbm.at[pl.ds(off, CHB)], wsems_c[u])

    def wait_write(j, u):
        off = base + j * CHB
        pltpu.make_async_copy(bufs_r[u], xr_hbm.at[pl.ds(off, CHB)],
                              wsems_r[u]).wait()
        pltpu.make_async_copy(bufs_c[u], xc_hbm.at[pl.ds(off, CHB)],
                              wsems_c[u]).wait()

    for k in range(G_AHEAD):
        fire_gather(k, k)

    ngrp = NFULL // G_SLOTS

    def body(t, carry):
        for u in range(G_SLOTS):
            j = t * G_SLOTS + u
            ua = (u + G_AHEAD) % G_SLOTS

            @pl.when(j + G_AHEAD < NFULL)
            def _():
                @pl.when(j >= G_AHEAD)
                def _():
                    wait_write(j - G_AHEAD, ua)

                fire_gather(j + G_AHEAD, ua)

            wait_gather(j, u)
            fire_write(j, u)
        return carry

    lax.fori_loop(0, ngrp, body, 0)
    for j in range(NFULL - G_SLOTS, NFULL):
        wait_write(j, j % G_SLOTS)

    cp_r = pltpu.async_copy(x_hbm.at[idx_rt.at[0]], brt, srt)
    cp_c = pltpu.async_copy(x_hbm.at[idx_ct.at[0]], bct, sct)
    cp_r.wait()
    cp_c.wait()
    off = base + TAIL_E_OFF
    pltpu.sync_copy(brt, xr_hbm.at[pl.ds(off, TAIL_E)])
    pltpu.sync_copy(bct, xc_hbm.at[pl.ds(off, TAIL_E)])


@jax.jit
def _gather(x, row_m, row_t, col_m, col_t):
    def wrapped(x_hbm, row_m_, row_t_, col_m_, col_t_, xr_hbm, xc_hbm,
                idx_rm, idx_rt, idx_cm, idx_ct, *rest):
        bufs_r = rest[0:G_SLOTS]
        bufs_c = rest[G_SLOTS:2 * G_SLOTS]
        brt, bct = rest[2 * G_SLOTS:2 * G_SLOTS + 2]
        sbase = 2 * G_SLOTS + 2
        gsems_r = rest[sbase:sbase + G_SLOTS]
        gsems_c = rest[sbase + G_SLOTS:sbase + 2 * G_SLOTS]
        wsems_r = rest[sbase + 2 * G_SLOTS:sbase + 3 * G_SLOTS]
        wsems_c = rest[sbase + 3 * G_SLOTS:sbase + 4 * G_SLOTS]
        srt, sct = rest[sbase + 4 * G_SLOTS:sbase + 4 * G_SLOTS + 2]
        _gather_body(x_hbm, row_m_, row_t_, col_m_, col_t_, xr_hbm, xc_hbm,
                     idx_rm, idx_rt, idx_cm, idx_ct,
                     bufs_r, bufs_c, brt, bct,
                     gsems_r, gsems_c, wsems_r, wsems_c, srt, sct)

    f = pl.kernel(
        wrapped,
        out_type=(
            jax.ShapeDtypeStruct((E_CHU, NODE_DIM), jnp.float32),
            jax.ShapeDtypeStruct((E_CHU, NODE_DIM), jnp.float32),
        ),
        mesh=_mesh(),
        scratch_types=(
            [pltpu.VMEM((NFULL, CHB), jnp.int32),
             pltpu.VMEM((1, TAIL_E), jnp.int32)] * 2
            + [pltpu.VMEM((CHB, NODE_DIM), jnp.float32)] * (2 * G_SLOTS)
            + [pltpu.VMEM((TAIL_E, NODE_DIM), jnp.float32)] * 2
            + [pltpu.SemaphoreType.DMA] * (4 * G_SLOTS + 2)
        ),
    )
    return f(x, row_m, row_t, col_m, col_t)


def _scatter_body(emb_hbm, col_s, zeros_hbm, out_hbm,
                  idx_m, b0, b1, b2, b3, shared,
                  s0, s1, s2, s3):
    c = lax.axis_index("c")
    s = lax.axis_index("s")
    wid = s * NC + c
    base = wid * PER_W
    r0 = s * ROWS_PER_SUB
    pltpu.sync_copy(zeros_hbm, shared.at[pl.ds(r0, ROWS_PER_SUB)])

    @pl.when(s == NS - 1)
    def _():
        pltpu.sync_copy(zeros_hbm.at[pl.ds(0, TAIL_ROWS)],
                        shared.at[pl.ds(TAIL_OFF, TAIL_ROWS)])

    pltpu.sync_copy(col_s.at[wid], idx_m)
    plsc.subcore_barrier()

    bufs = ((b0, s0), (b1, s1), (b2, s2), (b3, s3))

    def fire(j, k):
        b, sm = bufs[k]
        pltpu.async_copy(emb_hbm.at[pl.ds(base + j * S_CH, S_CH)], b, sm)

    def drain_add(j, k):
        b, sm = bufs[k]
        pltpu.make_async_copy(emb_hbm.at[pl.ds(base + j * S_CH, S_CH)], b,
                              sm).wait()
        pltpu.sync_copy(b, shared.at[idx_m.at[j]], add=True)

    for k in range(S_DEPTH):
        fire(k, k)

    ngrp = S_NCHUNK // S_DEPTH

    def body(t, carry):
        for u in range(S_DEPTH):
            j = t * S_DEPTH + u
            drain_add(j, u)

            @pl.when(j + S_DEPTH < S_NCHUNK)
            def _():
                fire(j + S_DEPTH, u)
        return carry

    lax.fori_loop(0, ngrp, body, 0)
    for u in range(S_NCHUNK - ngrp * S_DEPTH):
        drain_add(ngrp * S_DEPTH + u, u)

    plsc.subcore_barrier()
    pltpu.sync_copy(shared.at[pl.ds(r0, ROWS_PER_SUB)],
                    out_hbm.at[c, pl.ds(r0, ROWS_PER_SUB)])

    @pl.when(s == NS - 1)
    def _():
        pltpu.sync_copy(shared.at[pl.ds(TAIL_OFF, TAIL_ROWS)],
                        out_hbm.at[c, pl.ds(TAIL_OFF, TAIL_ROWS)])


@jax.jit
def _scatter(emb, col_s, zeros):
    f = pl.kernel(
        _scatter_body,
        out_type=jax.ShapeDtypeStruct((NC, N_NODES, NODE_DIM), jnp.float32),
        mesh=_mesh(),
        scratch_types=(
            [pltpu.VMEM((S_NCHUNK, S_CH), jnp.int32)]
            + [pltpu.VMEM((S_CH, NODE_DIM), jnp.float32)] * S_DEPTH
            + [pltpu.VMEM_SHARED((N_NODES, NODE_DIM), jnp.float32)]
            + [pltpu.SemaphoreType.DMA] * S_DEPTH
        ),
    )
    return f(emb, col_s, zeros)


def _edge_mlp_body(xr, xc, ea, w1a, w1b, w1c, b1, w2, b2, out):
    bf = jnp.bfloat16
    acc = jnp.dot(xr[...].astype(bf), w1a[...],
                  preferred_element_type=jnp.float32)
    acc += jnp.dot(xc[...].astype(bf), w1b[...],
                   preferred_element_type=jnp.float32)
    acc += jnp.dot(ea[...], w1c[...], preferred_element_type=jnp.float32)
    acc += b1[...]
    h = _softplus(acc)
    o = jnp.dot(h.astype(bf), w2[...],
                preferred_element_type=jnp.float32) + b2[...]
    out[...] = _softplus(o)


@jax.jit
def _edge_mlp(xr, xc, ea, w1a, w1b, w1c, b1, w2, b2):
    nblk = E_CHU // BE
    full = lambda shape: pl.BlockSpec(shape, lambda i: (0, 0))
    return pl.pallas_call(
        _edge_mlp_body,
        grid=(nblk,),
        in_specs=[
            pl.BlockSpec((BE, NODE_DIM), lambda i: (i, 0)),
            pl.BlockSpec((BE, NODE_DIM), lambda i: (i, 0)),
            pl.BlockSpec((BE, EDGE_DIM), lambda i: (i, 0)),
            full((NODE_DIM, 2 * NODE_DIM)),
            full((NODE_DIM, 2 * NODE_DIM)),
            full((EDGE_DIM, 2 * NODE_DIM)),
            full((1, 2 * NODE_DIM)),
            full((2 * NODE_DIM, NODE_DIM)),
            full((1, NODE_DIM)),
        ],
        out_specs=pl.BlockSpec((BE, NODE_DIM), lambda i: (i, 0)),
        out_shape=jax.ShapeDtypeStruct((E_CHU, NODE_DIM), jnp.float32),
        compiler_params=pltpu.CompilerParams(
            dimension_semantics=("parallel",)),
    )(xr, xc, ea, w1a, w1b, w1c, b1, w2, b2)


def _node_body(x, a0, a1, a2, a3, w3a, w3b, b3, out):
    ag = (a0[...] + a1[...]) + (a2[...] + a3[...])
    o = jnp.dot(x[...], w3a[...], preferred_element_type=jnp.float32)
    o += jnp.dot(ag, w3b[...], preferred_element_type=jnp.float32)
    o += b3[...]
    out[...] = _softplus(o) + x[...]


@jax.jit
def _node(x, a0, a1, a2, a3, w3a, w3b, b3):
    nblk = N_NODES // BN
    full = lambda shape: pl.BlockSpec(shape, lambda i: (0, 0))
    blk = pl.BlockSpec((BN, NODE_DIM), lambda i: (i, 0))
    return pl.pallas_call(
        _node_body,
        grid=(nblk,),
        in_specs=[
            blk, blk, blk, blk, blk,
            full((NODE_DIM, NODE_DIM)),
            full((NODE_DIM, NODE_DIM)),
            full((1, NODE_DIM)),
        ],
        out_specs=pl.BlockSpec((BN, NODE_DIM), lambda i: (i, 0)),
        out_shape=jax.ShapeDtypeStruct((N_NODES, NODE_DIM), jnp.float32),
        compiler_params=pltpu.CompilerParams(
            dimension_semantics=("parallel",)),
    )(x, a0, a1, a2, a3, w3a, w3b, b3)


def _split_idx(v):
    flat = v.astype(jnp.int32).reshape(NSPLIT, NW, PER_W)
    main = flat[:, :, :TAIL_E_OFF].reshape(NSPLIT, NW, NFULL, CHB)
    tail = flat[:, :, TAIL_E_OFF:].reshape(NSPLIT, NW, 1, TAIL_E)
    return main, tail


def kernel(x, edge_index, edge_attr, W1, b1, W2, b2, W3, b3):
    row_m, row_t = _split_idx(edge_index[0])
    col_m, col_t = _split_idx(edge_index[1])
    col_s = edge_index[1].astype(jnp.int32).reshape(NSPLIT, NW, S_NCHUNK, S_CH)
    bf = jnp.bfloat16
    w1a = W1[:NODE_DIM].astype(bf)
    w1b = W1[NODE_DIM:2 * NODE_DIM].astype(bf)
    w1c = W1[2 * NODE_DIM:].astype(bf)
    b1r = b1.reshape(1, -1)
    w2 = W2.astype(bf)
    b2r = b2.reshape(1, -1)
    zeros = jnp.zeros((ROWS_PER_SUB, NODE_DIM), jnp.float32)
    ea = edge_attr.astype(bf)

    parts = []
    for t in range(NSPLIT):
        xr, xc = _gather(x, row_m[t], row_t[t], col_m[t], col_t[t])
        emb = _edge_mlp(xr, xc, ea[t * E_CHU:(t + 1) * E_CHU],
                        w1a, w1b, w1c, b1r, w2, b2r)
        parts.append(_scatter(emb, col_s[t], zeros))
    return _node(x, parts[0][0], parts[0][1], parts[1][0], parts[1][1],
                 W3[:NODE_DIM], W3[NODE_DIM:], b3.reshape(1, -1))


# speedup vs baseline: 1.0379x; 1.0050x over previous
import jax
import jax.numpy as jnp
from jax import lax
from jax.experimental import pallas as pl
from jax.experimental.pallas import tpu as pltpu
from jax.experimental.pallas import tpu_sc as plsc

NODE_DIM = 128
EDGE_DIM = 16
N_NODES = 10000
N_EDGES = 320000

NC = 2
NS = 16
NW = NC * NS
NSPLIT = 2
E_CHU = N_EDGES // NSPLIT
PER_W = E_CHU // NW
CHB = 40
NFULL = PER_W // CHB
G_DEPTH = 4

ROWS_PER_SUB = 624
TAIL_ROWS = N_NODES - NS * ROWS_PER_SUB
TAIL_OFF = NS * ROWS_PER_SUB

S_CH = 40
S_NCHUNK = PER_W // S_CH
S_DEPTH = 4

BE = 4000
BN = 1000


def _mesh():
    return plsc.VectorSubcoreMesh(
        core_axis_name="c", subcore_axis_name="s", num_cores=NC, num_subcores=NS)


_NEG_LOG2E = -1.4426950408889634
_LN2 = 0.6931471805599453


def _softplus(v):
    e = jnp.exp2(jnp.abs(v) * _NEG_LOG2E)
    return jnp.maximum(v, 0.0) + jnp.log2(1.0 + e) * _LN2


def _gather_body(x_hbm, row_m, col_m, xr_hbm, xc_hbm,
                 idx_rm, idx_cm, bufs_r, bufs_c, sems_r, sems_c):
    c = lax.axis_index("c")
    s = lax.axis_index("s")
    wid = s * NC + c
    base = wid * PER_W
    pltpu.sync_copy(row_m.at[wid], idx_rm)
    pltpu.sync_copy(col_m.at[wid], idx_cm)

    def fire(j, u):
        pltpu.async_copy(x_hbm.at[idx_rm.at[j]], bufs_r[u], sems_r[u])
        pltpu.async_copy(x_hbm.at[idx_cm.at[j]], bufs_c[u], sems_c[u])

    def drain_write(j, u):
        pltpu.make_async_copy(x_hbm.at[idx_rm.at[j]], bufs_r[u],
                              sems_r[u]).wait()
        pltpu.make_async_copy(x_hbm.at[idx_cm.at[j]], bufs_c[u],
                              sems_c[u]).wait()
        off = base + j * CHB
        pltpu.sync_copy(bufs_r[u], xr_hbm.at[pl.ds(off, CHB)])
        pltpu.sync_copy(bufs_c[u], xc_hbm.at[pl.ds(off, CHB)])

    for k in range(G_DEPTH):
        fire(k, k)

    ngrp = NFULL // G_DEPTH

    def body(t, carry):
        for u in range(G_DEPTH):
            j = t * G_DEPTH + u
            drain_write(j, u)

            @pl.when(j + G_DEPTH < NFULL)
            def _():
                fire(j + G_DEPTH, u)
        return carry

    lax.fori_loop(0, ngrp, body, 0)
    for u in range(NFULL - (NFULL // G_DEPTH) * G_DEPTH):
        drain_write((NFULL // G_DEPTH) * G_DEPTH + u, u)


@jax.jit
def _gather(x, row_m, col_m):
    def wrapped(x_hbm, row_m_, col_m_, xr_hbm, xc_hbm,
                idx_rm, idx_cm, *rest):
        bufs_r = rest[0:G_DEPTH]
        bufs_c = rest[G_DEPTH:2 * G_DEPTH]
        sems_r = rest[2 * G_DEPTH:3 * G_DEPTH]
        sems_c = rest[3 * G_DEPTH:4 * G_DEPTH]
        _gather_body(x_hbm, row_m_, col_m_, xr_hbm, xc_hbm,
                     idx_rm, idx_cm, bufs_r, bufs_c, sems_r, sems_c)

    f = pl.kernel(
        wrapped,
        out_type=(
            jax.ShapeDtypeStruct((E_CHU, NODE_DIM), jnp.float32),
            jax.ShapeDtypeStruct((E_CHU, NODE_DIM), jnp.float32),
        ),
        mesh=_mesh(),
        scratch_types=(
            [pltpu.VMEM((NFULL, CHB), jnp.int32)] * 2
            + [pltpu.VMEM((CHB, NODE_DIM), jnp.float32)] * (2 * G_DEPTH)
            + [pltpu.SemaphoreType.DMA] * (2 * G_DEPTH)
        ),
    )
    return f(x, row_m, col_m)


def _scatter_body(emb_hbm, col_s, zeros_hbm, out_hbm,
                  idx_m, b0, b1, b2, b3, shared,
                  s0, s1, s2, s3):
    c = lax.axis_index("c")
    s = lax.axis_index("s")
    wid = s * NC + c
    base = wid * PER_W
    r0 = s * ROWS_PER_SUB
    pltpu.sync_copy(zeros_hbm, shared.at[pl.ds(r0, ROWS_PER_SUB)])

    @pl.when(s == NS - 1)
    def _():
        pltpu.sync_copy(zeros_hbm.at[pl.ds(0, TAIL_ROWS)],
                        shared.at[pl.ds(TAIL_OFF, TAIL_ROWS)])

    pltpu.sync_copy(col_s.at[wid], idx_m)
    plsc.subcore_barrier()

    bufs = ((b0, s0), (b1, s1), (b2, s2), (b3, s3))

    def fire(j, k):
        b, sm = bufs[k]
        pltpu.async_copy(emb_hbm.at[pl.ds(base + j * S_CH, S_CH)], b, sm)

    def drain_add(j, k):
        b, sm = bufs[k]
        pltpu.make_async_copy(emb_hbm.at[pl.ds(base + j * S_CH, S_CH)], b,
                              sm).wait()
        pltpu.sync_copy(b, shared.at[idx_m.at[j]], add=True)

    for k in range(S_DEPTH):
        fire(k, k)

    ngrp = S_NCHUNK // S_DEPTH

    def body(t, carry):
        for u in range(S_DEPTH):
            j = t * S_DEPTH + u
            drain_add(j, u)

            @pl.when(j + S_DEPTH < S_NCHUNK)
            def _():
                fire(j + S_DEPTH, u)
        return carry

    lax.fori_loop(0, ngrp, body, 0)
    for u in range(S_NCHUNK - ngrp * S_DEPTH):
        drain_add(ngrp * S_DEPTH + u, u)

    plsc.subcore_barrier()
    pltpu.sync_copy(shared.at[pl.ds(r0, ROWS_PER_SUB)],
                    out_hbm.at[c, pl.ds(r0, ROWS_PER_SUB)])

    @pl.when(s == NS - 1)
    def _():
        pltpu.sync_copy(shared.at[pl.ds(TAIL_OFF, TAIL_ROWS)],
                        out_hbm.at[c, pl.ds(TAIL_OFF, TAIL_ROWS)])


@jax.jit
def _scatter(emb, col_s, zeros):
    f = pl.kernel(
        _scatter_body,
        out_type=jax.ShapeDtypeStruct((NC, N_NODES, NODE_DIM), jnp.float32),
        mesh=_mesh(),
        scratch_types=(
            [pltpu.VMEM((S_NCHUNK, S_CH), jnp.int32)]
            + [pltpu.VMEM((S_CH, NODE_DIM), jnp.float32)] * S_DEPTH
            + [pltpu.VMEM_SHARED((N_NODES, NODE_DIM), jnp.float32)]
            + [pltpu.SemaphoreType.DMA] * S_DEPTH
        ),
    )
    return f(emb, col_s, zeros)


def _edge_mlp_body(xr, xc, ea, w1a, w1b, w1c, b1, w2, b2, out):
    bf = jnp.bfloat16
    acc = jnp.dot(xr[...].astype(bf), w1a[...],
                  preferred_element_type=jnp.float32)
    acc += jnp.dot(xc[...].astype(bf), w1b[...],
                   preferred_element_type=jnp.float32)
    acc += jnp.dot(ea[...], w1c[...], preferred_element_type=jnp.float32)
    acc += b1[...]
    h = _softplus(acc)
    o = jnp.dot(h.astype(bf), w2[...],
                preferred_element_type=jnp.float32) + b2[...]
    out[...] = _softplus(o)


@jax.jit
def _edge_mlp(xr, xc, ea, w1a, w1b, w1c, b1, w2, b2):
    nblk = E_CHU // BE
    full = lambda shape: pl.BlockSpec(shape, lambda i: (0, 0))
    return pl.pallas_call(
        _edge_mlp_body,
        grid=(nblk,),
        in_specs=[
            pl.BlockSpec((BE, NODE_DIM), lambda i: (i, 0)),
            pl.BlockSpec((BE, NODE_DIM), lambda i: (i, 0)),
            pl.BlockSpec((BE, EDGE_DIM), lambda i: (i, 0)),
            full((NODE_DIM, 2 * NODE_DIM)),
            full((NODE_DIM, 2 * NODE_DIM)),
            full((EDGE_DIM, 2 * NODE_DIM)),
            full((1, 2 * NODE_DIM)),
            full((2 * NODE_DIM, NODE_DIM)),
            full((1, NODE_DIM)),
        ],
        out_specs=pl.BlockSpec((BE, NODE_DIM), lambda i: (i, 0)),
        out_shape=jax.ShapeDtypeStruct((E_CHU, NODE_DIM), jnp.float32),
        compiler_params=pltpu.CompilerParams(
            dimension_semantics=("parallel",)),
    )(xr, xc, ea, w1a, w1b, w1c, b1, w2, b2)


def _node_body(x, a0, a1, a2, a3, w3a, w3b, b3, out):
    ag = (a0[...] + a1[...]) + (a2[...] + a3[...])
    o = jnp.dot(x[...], w3a[...], preferred_element_type=jnp.float32)
    o += jnp.dot(ag, w3b[...], preferred_element_type=jnp.float32)
    o += b3[...]
    out[...] = _softplus(o) + x[...]


@jax.jit
def _node(x, a0, a1, a2, a3, w3a, w3b, b3):
    nblk = N_NODES // BN
    full = lambda shape: pl.BlockSpec(shape, lambda i: (0, 0))
    blk = pl.BlockSpec((BN, NODE_DIM), lambda i: (i, 0))
    return pl.pallas_call(
        _node_body,
        grid=(nblk,),
        in_specs=[
            blk, blk, blk, blk, blk,
            full((NODE_DIM, NODE_DIM)),
            full((NODE_DIM, NODE_DIM)),
            full((1, NODE_DIM)),
        ],
        out_specs=pl.BlockSpec((BN, NODE_DIM), lambda i: (i, 0)),
        out_shape=jax.ShapeDtypeStruct((N_NODES, NODE_DIM), jnp.float32),
        compiler_params=pltpu.CompilerParams(
            dimension_semantics=("parallel",)),
    )(x, a0, a1, a2, a3, w3a, w3b, b3)


def kernel(x, edge_index, edge_attr, W1, b1, W2, b2, W3, b3):
    row_m = edge_index[0].astype(jnp.int32).reshape(NSPLIT, NW, NFULL, CHB)
    col_m = edge_index[1].astype(jnp.int32).reshape(NSPLIT, NW, NFULL, CHB)
    bf = jnp.bfloat16
    w1a = W1[:NODE_DIM].astype(bf)
    w1b = W1[NODE_DIM:2 * NODE_DIM].astype(bf)
    w1c = W1[2 * NODE_DIM:].astype(bf)
    b1r = b1.reshape(1, -1)
    w2 = W2.astype(bf)
    b2r = b2.reshape(1, -1)
    zeros = jnp.zeros((ROWS_PER_SUB, NODE_DIM), jnp.float32)
    ea = edge_attr.astype(bf)

    parts = []
    for t in range(NSPLIT):
        xr, xc = _gather(x, row_m[t], col_m[t])
        emb = _edge_mlp(xr, xc, ea[t * E_CHU:(t + 1) * E_CHU],
                        w1a, w1b, w1c, b1r, w2, b2r)
        parts.append(_scatter(emb, col_m[t], zeros))
    return _node(x, parts[0][0], parts[0][1], parts[1][0], parts[1][1],
                 W3[:NODE_DIM], W3[NODE_DIM:], b3.reshape(1, -1))


# speedup vs baseline: 1.0470x; 1.0087x over previous
import jax
import jax.numpy as jnp
from jax import lax
from jax.experimental import pallas as pl
from jax.experimental.pallas import tpu as pltpu
from jax.experimental.pallas import tpu_sc as plsc

NODE_DIM = 128
EDGE_DIM = 16
N_NODES = 10000
N_EDGES = 320000

NC = 2
NS = 16
NW = NC * NS
NSPLIT = 2
E_CHU = N_EDGES // NSPLIT
PER_W = E_CHU // NW
CHB = 40
NFULL = PER_W // CHB
G_DEPTH = 4

ROWS_PER_SUB = 624
TAIL_ROWS = N_NODES - NS * ROWS_PER_SUB
TAIL_OFF = NS * ROWS_PER_SUB

S_CH = 40
S_NCHUNK = PER_W // S_CH
S_DEPTH = 4

BE = 8000
BN = 1000


def _mesh():
    return plsc.VectorSubcoreMesh(
        core_axis_name="c", subcore_axis_name="s", num_cores=NC, num_subcores=NS)


_NEG_LOG2E = -1.4426950408889634
_LN2 = 0.6931471805599453


def _softplus(v):
    e = jnp.exp2(jnp.abs(v) * _NEG_LOG2E)
    return jnp.maximum(v, 0.0) + jnp.log2(1.0 + e) * _LN2


def _gather_body(x_hbm, row_m, col_m, xr_hbm, xc_hbm,
                 idx_rm, idx_cm, bufs_r, bufs_c, sems_r, sems_c):
    c = lax.axis_index("c")
    s = lax.axis_index("s")
    wid = s * NC + c
    base = wid * PER_W
    pltpu.sync_copy(row_m.at[wid], idx_rm)
    pltpu.sync_copy(col_m.at[wid], idx_cm)

    def fire(j, u):
        pltpu.async_copy(x_hbm.at[idx_rm.at[j]], bufs_r[u], sems_r[u])
        pltpu.async_copy(x_hbm.at[idx_cm.at[j]], bufs_c[u], sems_c[u])

    def drain_write(j, u):
        pltpu.make_async_copy(x_hbm.at[idx_rm.at[j]], bufs_r[u],
                              sems_r[u]).wait()
        pltpu.make_async_copy(x_hbm.at[idx_cm.at[j]], bufs_c[u],
                              sems_c[u]).wait()
        off = base + j * CHB
        pltpu.sync_copy(bufs_r[u], xr_hbm.at[pl.ds(off, CHB)])
        pltpu.sync_copy(bufs_c[u], xc_hbm.at[pl.ds(off, CHB)])

    for k in range(G_DEPTH):
        fire(k, k)

    ngrp = NFULL // G_DEPTH

    def body(t, carry):
        for u in range(G_DEPTH):
            j = t * G_DEPTH + u
            drain_write(j, u)

            @pl.when(j + G_DEPTH < NFULL)
            def _():
                fire(j + G_DEPTH, u)
        return carry

    lax.fori_loop(0, ngrp, body, 0)
    for u in range(NFULL - (NFULL // G_DEPTH) * G_DEPTH):
        drain_write((NFULL // G_DEPTH) * G_DEPTH + u, u)


@jax.jit
def _gather(x, row_m, col_m):
    def wrapped(x_hbm, row_m_, col_m_, xr_hbm, xc_hbm,
                idx_rm, idx_cm, *rest):
        bufs_r = rest[0:G_DEPTH]
        bufs_c = rest[G_DEPTH:2 * G_DEPTH]
        sems_r = rest[2 * G_DEPTH:3 * G_DEPTH]
        sems_c = rest[3 * G_DEPTH:4 * G_DEPTH]
        _gather_body(x_hbm, row_m_, col_m_, xr_hbm, xc_hbm,
                     idx_rm, idx_cm, bufs_r, bufs_c, sems_r, sems_c)

    f = pl.kernel(
        wrapped,
        out_type=(
            jax.ShapeDtypeStruct((E_CHU, NODE_DIM), jnp.float32),
            jax.ShapeDtypeStruct((E_CHU, NODE_DIM), jnp.float32),
        ),
        mesh=_mesh(),
        scratch_types=(
            [pltpu.VMEM((NFULL, CHB), jnp.int32)] * 2
            + [pltpu.VMEM((CHB, NODE_DIM), jnp.float32)] * (2 * G_DEPTH)
            + [pltpu.SemaphoreType.DMA] * (2 * G_DEPTH)
        ),
    )
    return f(x, row_m, col_m)


def _scatter_body(emb_hbm, col_s, zeros_hbm, out_hbm,
                  idx_m, b0, b1, b2, b3, shared,
                  s0, s1, s2, s3):
    c = lax.axis_index("c")
    s = lax.axis_index("s")
    wid = s * NC + c
    base = wid * PER_W
    r0 = s * ROWS_PER_SUB
    pltpu.sync_copy(zeros_hbm, shared.at[pl.ds(r0, ROWS_PER_SUB)])

    @pl.when(s == NS - 1)
    def _():
        pltpu.sync_copy(zeros_hbm.at[pl.ds(0, TAIL_ROWS)],
                        shared.at[pl.ds(TAIL_OFF, TAIL_ROWS)])

    pltpu.sync_copy(col_s.at[wid], idx_m)
    plsc.subcore_barrier()

    bufs = ((b0, s0), (b1, s1), (b2, s2), (b3, s3))

    def fire(j, k):
        b, sm = bufs[k]
        pltpu.async_copy(emb_hbm.at[pl.ds(base + j * S_CH, S_CH)], b, sm)

    def drain_add(j, k):
        b, sm = bufs[k]
        pltpu.make_async_copy(emb_hbm.at[pl.ds(base + j * S_CH, S_CH)], b,
                              sm).wait()
        pltpu.sync_copy(b, shared.at[idx_m.at[j]], add=True)

    for k in range(S_DEPTH):
        fire(k, k)

    ngrp = S_NCHUNK // S_DEPTH

    def body(t, carry):
        for u in range(S_DEPTH):
            j = t * S_DEPTH + u
            drain_add(j, u)

            @pl.when(j + S_DEPTH < S_NCHUNK)
            def _():
                fire(j + S_DEPTH, u)
        return carry

    lax.fori_loop(0, ngrp, body, 0)
    for u in range(S_NCHUNK - ngrp * S_DEPTH):
        drain_add(ngrp * S_DEPTH + u, u)

    plsc.subcore_barrier()
    pltpu.sync_copy(shared.at[pl.ds(r0, ROWS_PER_SUB)],
                    out_hbm.at[c, pl.ds(r0, ROWS_PER_SUB)])

    @pl.when(s == NS - 1)
    def _():
        pltpu.sync_copy(shared.at[pl.ds(TAIL_OFF, TAIL_ROWS)],
                        out_hbm.at[c, pl.ds(TAIL_OFF, TAIL_ROWS)])


@jax.jit
def _scatter(emb, col_s, zeros):
    f = pl.kernel(
        _scatter_body,
        out_type=jax.ShapeDtypeStruct((NC, N_NODES, NODE_DIM), jnp.float32),
        mesh=_mesh(),
        scratch_types=(
            [pltpu.VMEM((S_NCHUNK, S_CH), jnp.int32)]
            + [pltpu.VMEM((S_CH, NODE_DIM), jnp.float32)] * S_DEPTH
            + [pltpu.VMEM_SHARED((N_NODES, NODE_DIM), jnp.float32)]
            + [pltpu.SemaphoreType.DMA] * S_DEPTH
        ),
    )
    return f(emb, col_s, zeros)


def _edge_mlp_body(xr, xc, ea, w1a, w1b, w1c, b1, w2, b2, out):
    bf = jnp.bfloat16
    acc = jnp.dot(xr[...].astype(bf), w1a[...],
                  preferred_element_type=jnp.float32)
    acc += jnp.dot(xc[...].astype(bf), w1b[...],
                   preferred_element_type=jnp.float32)
    acc += jnp.dot(ea[...], w1c[...], preferred_element_type=jnp.float32)
    acc += b1[...]
    h = _softplus(acc)
    o = jnp.dot(h.astype(bf), w2[...],
                preferred_element_type=jnp.float32) + b2[...]
    out[...] = _softplus(o)


@jax.jit
def _edge_mlp(xr, xc, ea, w1a, w1b, w1c, b1, w2, b2):
    nblk = E_CHU // BE
    full = lambda shape: pl.BlockSpec(shape, lambda i: (0, 0))
    return pl.pallas_call(
        _edge_mlp_body,
        grid=(nblk,),
        in_specs=[
            pl.BlockSpec((BE, NODE_DIM), lambda i: (i, 0)),
            pl.BlockSpec((BE, NODE_DIM), lambda i: (i, 0)),
            pl.BlockSpec((BE, EDGE_DIM), lambda i: (i, 0)),
            full((NODE_DIM, 2 * NODE_DIM)),
            full((NODE_DIM, 2 * NODE_DIM)),
            full((EDGE_DIM, 2 * NODE_DIM)),
            full((1, 2 * NODE_DIM)),
            full((2 * NODE_DIM, NODE_DIM)),
            full((1, NODE_DIM)),
        ],
        out_specs=pl.BlockSpec((BE, NODE_DIM), lambda i: (i, 0)),
        out_shape=jax.ShapeDtypeStruct((E_CHU, NODE_DIM), jnp.float32),
        compiler_params=pltpu.CompilerParams(
            dimension_semantics=("parallel",)),
    )(xr, xc, ea, w1a, w1b, w1c, b1, w2, b2)


def _node_body(x, a0, a1, a2, a3, w3a, w3b, b3, out):
    ag = (a0[...] + a1[...]) + (a2[...] + a3[...])
    o = jnp.dot(x[...], w3a[...], preferred_element_type=jnp.float32)
    o += jnp.dot(ag, w3b[...], preferred_element_type=jnp.float32)
    o += b3[...]
    out[...] = _softplus(o) + x[...]


@jax.jit
def _node(x, a0, a1, a2, a3, w3a, w3b, b3):
    nblk = N_NODES // BN
    full = lambda shape: pl.BlockSpec(shape, lambda i: (0, 0))
    blk = pl.BlockSpec((BN, NODE_DIM), lambda i: (i, 0))
    return pl.pallas_call(
        _node_body,
        grid=(nblk,),
        in_specs=[
            blk, blk, blk, blk, blk,
            full((NODE_DIM, NODE_DIM)),
            full((NODE_DIM, NODE_DIM)),
            full((1, NODE_DIM)),
        ],
        out_specs=pl.BlockSpec((BN, NODE_DIM), lambda i: (i, 0)),
        out_shape=jax.ShapeDtypeStruct((N_NODES, NODE_DIM), jnp.float32),
        compiler_params=pltpu.CompilerParams(
            dimension_semantics=("parallel",)),
    )(x, a0, a1, a2, a3, w3a, w3b, b3)


def kernel(x, edge_index, edge_attr, W1, b1, W2, b2, W3, b3):
    row_m = edge_index[0].astype(jnp.int32).reshape(NSPLIT, NW, NFULL, CHB)
    col_m = edge_index[1].astype(jnp.int32).reshape(NSPLIT, NW, NFULL, CHB)
    bf = jnp.bfloat16
    w1a = W1[:NODE_DIM].astype(bf)
    w1b = W1[NODE_DIM:2 * NODE_DIM].astype(bf)
    w1c = W1[2 * NODE_DIM:].astype(bf)
    b1r = b1.reshape(1, -1)
    w2 = W2.astype(bf)
    b2r = b2.reshape(1, -1)
    zeros = jnp.zeros((ROWS_PER_SUB, NODE_DIM), jnp.float32)
    ea = edge_attr.astype(bf)

    parts = []
    for t in range(NSPLIT):
        xr, xc = _gather(x, row_m[t], col_m[t])
        emb = _edge_mlp(xr, xc, ea[t * E_CHU:(t + 1) * E_CHU],
                        w1a, w1b, w1c, b1r, w2, b2r)
        parts.append(_scatter(emb, col_m[t], zeros))
    return _node(x, parts[0][0], parts[0][1], parts[1][0], parts[1][1],
                 W3[:NODE_DIM], W3[NODE_DIM:], b3.reshape(1, -1))
